# baseline (speedup 1.0000x reference)
import functools

import jax
import jax.numpy as jnp
import numpy as np
from jax import lax
from jax.experimental import pallas as pl
from jax.experimental.pallas import tpu as pltpu
from jax.experimental.pallas import tpu_sc as plsc

_NUM = 16


def _tf2x32_block(k1, k2, x0, x1):
    r0 = (13, 15, 26, 6)
    r1 = (17, 29, 16, 24)

    def rotl(x, d):
        return ((x << np.uint32(d)) | (x >> np.uint32(32 - d))).astype(
            np.uint32)

    def rounds(x, rots):
        for r in rots:
            a = (x[0] + x[1]).astype(np.uint32)
            b = (a ^ rotl(x[1], r)).astype(np.uint32)
            x = [a, b]
        return x

    ks = [k1, k2, np.uint32(k1 ^ k2 ^ np.uint32(0x1BD11BDA))]
    x = [(x0 + ks[0]).astype(np.uint32), (x1 + ks[1]).astype(np.uint32)]
    for i in range(5):
        x = rounds(x, r0 if i % 2 == 0 else r1)
        x = [(x[0] + ks[(i + 1) % 3]).astype(np.uint32),
             (x[1] + ks[(i + 2) % 3] + np.uint32(i + 1)).astype(np.uint32)]
    return x


def _tf_fold_in(key, d):
    seed = np.array([(d >> 32) & 0xFFFFFFFF, d & 0xFFFFFFFF], np.uint32)
    y0, y1 = _tf2x32_block(key[0], key[1], seed[:1], seed[1:])
    return np.concatenate([y0, y1])


def _np_permutation(key, n):
    num_rounds = max(1, int(np.ceil(3 * np.log(max(1, n)) /
                                    np.log(np.iinfo(np.uint32).max))))
    x = np.arange(n, dtype=np.int32)
    for _ in range(num_rounds):
        b1, b2 = _tf2x32_block(key[0], key[1], np.zeros(2, np.uint32),
                               np.arange(2, dtype=np.uint32))
        key, sub = np.stack([b1, b2], axis=1)
        s1, s2 = _tf2x32_block(sub[0], sub[1], np.zeros(n, np.uint32),
                               np.arange(n, dtype=np.uint32))
        x = x[np.argsort((s1 ^ s2).astype(np.uint32), kind="stable")]
    return x


@functools.lru_cache(maxsize=None)
def _pools(h, w):
    key1 = np.array([0, 1], np.uint32)
    pool_x = _np_permutation(_tf_fold_in(key1, 0), h)
    pool_y = _np_permutation(_tf_fold_in(key1, 1), w)
    return pool_x, pool_y


def _make_sc_kernel(b, c, h, w, planes_per_worker):
    mesh = plsc.VectorSubcoreMesh(core_axis_name="c", subcore_axis_name="s")
    nc = plsc.get_sparse_core_info().num_cores
    pw = planes_per_worker
    half = h // 2

    @functools.partial(
        pl.kernel,
        out_type=jax.ShapeDtypeStruct((b, c, h, w), jnp.float32),
        mesh=mesh,
        compiler_params=pltpu.CompilerParams(needs_layout_passes=False),
        scratch_types=[
            pltpu.VMEM((h, w), jnp.float32),
            pltpu.VMEM((h, w), jnp.float32),
            pltpu.VMEM((4, _NUM), jnp.int32),
            pltpu.SemaphoreType.DMA,
            pltpu.SemaphoreType.DMA,
            pltpu.SemaphoreType.DMA,
            pltpu.SemaphoreType.DMA,
            pltpu.SemaphoreType.DMA,
            pltpu.SemaphoreType.DMA,
            pltpu.SemaphoreType.DMA,
            pltpu.SemaphoreType.DMA,
        ],
    )
    def sc_kernel(x_hbm, idx_hbm, out_hbm, buf0, buf1, idx_v,
                  ra0, rb0, ra1, rb1, wa0, wb0, wa1, wb1):
        wid = lax.axis_index("s") * nc + lax.axis_index("c")
        base = wid * pw
        pltpu.sync_copy(idx_hbm, idx_v)
        pxv = idx_v[0, :]
        pyv = idx_v[1, :]
        sxv = idx_v[2, :]
        syv = idx_v[3, :]
        bufs = (buf0, buf1)
        rsems = ((ra0, rb0), (ra1, rb1))
        wsems = ((wa0, wb0), (wa1, wb1))
        reads = [None, None]
        writes = [None, None]

        def plane_copy(src, dst, sems):
            cps = (pltpu.make_async_copy(src.at[pl.ds(0, half)],
                                         dst.at[pl.ds(0, half)], sems[0]),
                   pltpu.make_async_copy(src.at[pl.ds(half, half)],
                                         dst.at[pl.ds(half, half)], sems[1]))
            cps[0].start()
            cps[1].start()
            return cps

        for p in range(pw + 1):
            if p < pw:
                k = p % 2
                if writes[k] is not None:
                    writes[k][0].wait()
                    writes[k][1].wait()
                pi = base + p
                reads[k] = plane_copy(x_hbm.at[pi // c, pi % c], bufs[k],
                                      rsems[k])
            if p >= 1:
                k = (p - 1) % 2
                reads[k][0].wait()
                reads[k][1].wait()
                vals = plsc.load_gather(bufs[k], [sxv, syv])
                plsc.store_scatter(bufs[k], [pxv, pyv], vals)
                pi = base + p - 1
                writes[k] = plane_copy(bufs[k], out_hbm.at[pi // c, pi % c],
                                       wsems[k])
        for k in range(2):
            writes[k][0].wait()
            writes[k][1].wait()

    return sc_kernel


def kernel(x):
    b, c, h, w = x.shape
    pool_x, pool_y = _pools(h, w)
    px = jnp.asarray(pool_x[:_NUM])
    py = jnp.asarray(pool_y[:_NUM])
    sx = pool_x[-_NUM:]
    sy = pool_y[-_NUM:]
    idx = jnp.asarray(
        np.stack([pool_x[:_NUM], pool_y[:_NUM], sx, sy]).astype(np.int32))

    p = b * c
    info = plsc.get_sparse_core_info()
    nw = info.num_cores * info.num_subcores
    assert p % nw == 0 and h % 2 == 0
    out = _make_sc_kernel(b, c, h, w, p // nw)(x, idx)
    return out, (px, py)


# speedup vs baseline: 1.0028x; 1.0028x over previous
import functools

import jax
import jax.numpy as jnp
import numpy as np
from jax import lax
from jax.experimental import pallas as pl
from jax.experimental.pallas import tpu as pltpu
from jax.experimental.pallas import tpu_sc as plsc

_NUM = 16


def _tf2x32_block(k1, k2, x0, x1):
    r0 = (13, 15, 26, 6)
    r1 = (17, 29, 16, 24)

    def rotl(x, d):
        return ((x << np.uint32(d)) | (x >> np.uint32(32 - d))).astype(
            np.uint32)

    def rounds(x, rots):
        for r in rots:
            a = (x[0] + x[1]).astype(np.uint32)
            b = (a ^ rotl(x[1], r)).astype(np.uint32)
            x = [a, b]
        return x

    ks = [k1, k2, np.uint32(k1 ^ k2 ^ np.uint32(0x1BD11BDA))]
    x = [(x0 + ks[0]).astype(np.uint32), (x1 + ks[1]).astype(np.uint32)]
    for i in range(5):
        x = rounds(x, r0 if i % 2 == 0 else r1)
        x = [(x[0] + ks[(i + 1) % 3]).astype(np.uint32),
             (x[1] + ks[(i + 2) % 3] + np.uint32(i + 1)).astype(np.uint32)]
    return x


def _tf_fold_in(key, d):
    seed = np.array([(d >> 32) & 0xFFFFFFFF, d & 0xFFFFFFFF], np.uint32)
    y0, y1 = _tf2x32_block(key[0], key[1], seed[:1], seed[1:])
    return np.concatenate([y0, y1])


def _np_permutation(key, n):
    num_rounds = max(1, int(np.ceil(3 * np.log(max(1, n)) /
                                    np.log(np.iinfo(np.uint32).max))))
    x = np.arange(n, dtype=np.int32)
    for _ in range(num_rounds):
        b1, b2 = _tf2x32_block(key[0], key[1], np.zeros(2, np.uint32),
                               np.arange(2, dtype=np.uint32))
        key, sub = np.stack([b1, b2], axis=1)
        s1, s2 = _tf2x32_block(sub[0], sub[1], np.zeros(n, np.uint32),
                               np.arange(n, dtype=np.uint32))
        x = x[np.argsort((s1 ^ s2).astype(np.uint32), kind="stable")]
    return x


@functools.lru_cache(maxsize=None)
def _pools(h, w):
    key1 = np.array([0, 1], np.uint32)
    pool_x = _np_permutation(_tf_fold_in(key1, 0), h)
    pool_y = _np_permutation(_tf_fold_in(key1, 1), w)
    return pool_x, pool_y


def _make_sc_kernel(b, c, h, w, planes_per_worker):
    mesh = plsc.VectorSubcoreMesh(core_axis_name="c", subcore_axis_name="s")
    nc = plsc.get_sparse_core_info().num_cores
    pw = planes_per_worker

    @functools.partial(
        pl.kernel,
        out_type=jax.ShapeDtypeStruct((b, c, h, w), jnp.float32),
        mesh=mesh,
        compiler_params=pltpu.CompilerParams(needs_layout_passes=False),
        scratch_types=[
            pltpu.VMEM((h, w), jnp.float32),
            pltpu.VMEM((h, w), jnp.float32),
            pltpu.VMEM((4, _NUM), jnp.int32),
            pltpu.SemaphoreType.DMA,
            pltpu.SemaphoreType.DMA,
            pltpu.SemaphoreType.DMA,
            pltpu.SemaphoreType.DMA,
        ],
    )
    def sc_kernel(x_hbm, idx_hbm, out_hbm, buf0, buf1, idx_v,
                  rsem0, rsem1, wsem0, wsem1):
        wid = lax.axis_index("s") * nc + lax.axis_index("c")
        base = wid * pw
        pltpu.sync_copy(idx_hbm, idx_v)
        pxv = idx_v[0, :]
        pyv = idx_v[1, :]
        sxv = idx_v[2, :]
        syv = idx_v[3, :]
        bufs = (buf0, buf1)
        rsems = (rsem0, rsem1)
        wsems = (wsem0, wsem1)
        reads = [None, None]
        writes = [None, None]
        for p in range(pw + 1):
            if p < pw:
                k = p % 2
                if writes[k] is not None:
                    writes[k].wait()
                pi = base + p
                reads[k] = pltpu.make_async_copy(
                    x_hbm.at[pi // c, pi % c], bufs[k], rsems[k])
                reads[k].start()
            if p >= 1:
                k = (p - 1) % 2
                reads[k].wait()
                vals = plsc.load_gather(bufs[k], [sxv, syv])
                plsc.store_scatter(bufs[k], [pxv, pyv], vals)
                pi = base + p - 1
                writes[k] = pltpu.make_async_copy(
                    bufs[k], out_hbm.at[pi // c, pi % c], wsems[k])
                writes[k].start()
        writes[(pw - 1) % 2].wait()
        writes[pw % 2].wait()

    return sc_kernel


def kernel(x):
    b, c, h, w = x.shape
    pool_x, pool_y = _pools(h, w)
    px = jnp.asarray(pool_x[:_NUM])
    py = jnp.asarray(pool_y[:_NUM])
    sx = pool_x[-_NUM:]
    sy = pool_y[-_NUM:]
    idx = jnp.asarray(
        np.stack([pool_x[:_NUM], pool_y[:_NUM], sx, sy]).astype(np.int32))

    p = b * c
    info = plsc.get_sparse_core_info()
    nw = info.num_cores * info.num_subcores
    assert p % nw == 0 and h % 2 == 0
    out = _make_sc_kernel(b, c, h, w, p // nw)(x, idx)
    return out, (px, py)


# speedup vs baseline: 1.0333x; 1.0304x over previous
import functools

import jax
import jax.numpy as jnp
import numpy as np
from jax import lax
from jax.experimental import pallas as pl
from jax.experimental.pallas import tpu as pltpu
from jax.experimental.pallas import tpu_sc as plsc

_NUM = 16


def _tf2x32_block(k1, k2, x0, x1):
    r0 = (13, 15, 26, 6)
    r1 = (17, 29, 16, 24)

    def rotl(x, d):
        return ((x << np.uint32(d)) | (x >> np.uint32(32 - d))).astype(
            np.uint32)

    def rounds(x, rots):
        for r in rots:
            a = (x[0] + x[1]).astype(np.uint32)
            b = (a ^ rotl(x[1], r)).astype(np.uint32)
            x = [a, b]
        return x

    ks = [k1, k2, np.uint32(k1 ^ k2 ^ np.uint32(0x1BD11BDA))]
    x = [(x0 + ks[0]).astype(np.uint32), (x1 + ks[1]).astype(np.uint32)]
    for i in range(5):
        x = rounds(x, r0 if i % 2 == 0 else r1)
        x = [(x[0] + ks[(i + 1) % 3]).astype(np.uint32),
             (x[1] + ks[(i + 2) % 3] + np.uint32(i + 1)).astype(np.uint32)]
    return x


def _tf_fold_in(key, d):
    seed = np.array([(d >> 32) & 0xFFFFFFFF, d & 0xFFFFFFFF], np.uint32)
    y0, y1 = _tf2x32_block(key[0], key[1], seed[:1], seed[1:])
    return np.concatenate([y0, y1])


def _np_permutation(key, n):
    num_rounds = max(1, int(np.ceil(3 * np.log(max(1, n)) /
                                    np.log(np.iinfo(np.uint32).max))))
    x = np.arange(n, dtype=np.int32)
    for _ in range(num_rounds):
        b1, b2 = _tf2x32_block(key[0], key[1], np.zeros(2, np.uint32),
                               np.arange(2, dtype=np.uint32))
        key, sub = np.stack([b1, b2], axis=1)
        s1, s2 = _tf2x32_block(sub[0], sub[1], np.zeros(n, np.uint32),
                               np.arange(n, dtype=np.uint32))
        x = x[np.argsort((s1 ^ s2).astype(np.uint32), kind="stable")]
    return x


@functools.lru_cache(maxsize=None)
def _pools(h, w):
    key1 = np.array([0, 1], np.uint32)
    pool_x = _np_permutation(_tf_fold_in(key1, 0), h)
    pool_y = _np_permutation(_tf_fold_in(key1, 1), w)
    return pool_x, pool_y


def _make_sc_kernel(b, c, h, w, planes_per_worker):
    mesh = plsc.VectorSubcoreMesh(core_axis_name="c", subcore_axis_name="s")
    nc = plsc.get_sparse_core_info().num_cores
    pw = planes_per_worker

    @functools.partial(
        pl.kernel,
        out_type=jax.ShapeDtypeStruct((b, c, h, w), jnp.float32),
        mesh=mesh,
        compiler_params=pltpu.CompilerParams(needs_layout_passes=False),
        scratch_types=[
            pltpu.VMEM((h, w), jnp.float32),
            pltpu.VMEM((h, w), jnp.float32),
            pltpu.VMEM((4, _NUM), jnp.int32),
            pltpu.SemaphoreType.DMA,
            pltpu.SemaphoreType.DMA,
            pltpu.SemaphoreType.DMA,
            pltpu.SemaphoreType.DMA,
        ],
    )
    def sc_kernel(x_hbm, idx_hbm, out_hbm, buf0, buf1, idx_v,
                  rsem0, rsem1, wsem0, wsem1):
        wid = lax.axis_index("s") * nc + lax.axis_index("c")
        base = wid * pw
        pltpu.sync_copy(idx_hbm, idx_v)
        pxv = idx_v[0, :]
        pyv = idx_v[1, :]
        sxv = idx_v[2, :]
        syv = idx_v[3, :]
        bufs = (buf0, buf1)
        rsems = (rsem0, rsem1)
        wsems = (wsem0, wsem1)

        def read_cp(p, k):
            pi = base + p
            return pltpu.make_async_copy(
                x_hbm.at[pi // c, pi % c], bufs[k], rsems[k])

        def write_cp(p, k):
            pi = base + p
            return pltpu.make_async_copy(
                bufs[k], out_hbm.at[pi // c, pi % c], wsems[k])

        def fix_and_write(p, k):
            read_cp(p, k).wait()
            vals = plsc.load_gather(bufs[k], [sxv, syv])
            plsc.store_scatter(bufs[k], [pxv, pyv], vals)
            write_cp(p, k).start()

        read_cp(0, 0).start()
        read_cp(1, 1).start()

        @pl.loop(0, pw - 2, step=2)
        def _body(p):
            fix_and_write(p, 0)
            write_cp(p, 0).wait()
            read_cp(p + 2, 0).start()
            fix_and_write(p + 1, 1)
            write_cp(p + 1, 1).wait()
            read_cp(p + 3, 1).start()

        fix_and_write(pw - 2, 0)
        write_cp(pw - 2, 0).wait()
        fix_and_write(pw - 1, 1)
        write_cp(pw - 1, 1).wait()

    return sc_kernel


def kernel(x):
    b, c, h, w = x.shape
    pool_x, pool_y = _pools(h, w)
    px = jnp.asarray(pool_x[:_NUM])
    py = jnp.asarray(pool_y[:_NUM])
    sx = pool_x[-_NUM:]
    sy = pool_y[-_NUM:]
    idx = jnp.asarray(
        np.stack([pool_x[:_NUM], pool_y[:_NUM], sx, sy]).astype(np.int32))

    p = b * c
    info = plsc.get_sparse_core_info()
    nw = info.num_cores * info.num_subcores
    assert p % nw == 0 and (p // nw) % 2 == 0 and p // nw >= 4
    out = _make_sc_kernel(b, c, h, w, p // nw)(x, idx)
    return out, (px, py)
